# Initial kernel scaffold; baseline (speedup 1.0000x reference)
import jax
import jax.numpy as jnp
from jax.experimental import pallas as pl


def kernel(x, edge_index, Wz, bz, Wr, br, Wh, bh, LzW, Lzb, LrW, Lrb, LhW, Lhb, att, WL, bL):
    raise NotImplementedError("write your pallas kernel here")


# speedup vs baseline: 25.0857x; 25.0857x over previous
import functools

import jax
import jax.numpy as jnp
from jax import lax
from jax.experimental import pallas as pl
from jax.experimental.pallas import tpu as pltpu
import jax.experimental.pallas.tpu_sc as plsc

N = 10000
F_IN = 128
OUT = 16
T = 12
E = 320000

NC = 2
NS = 16
NW = NC * NS

_NSH = 632
N_PAD = _NSH * NS
_TRASH = N_PAD

_CH1 = 128
_CPW1 = 79
_EPW1 = _CH1 * _CPW1
_EP1 = _EPW1 * NW
_DW = 16


def _zero_shard(buf_v, shared, s, width):
    del width
    for j in range(4):
        pltpu.sync_copy(buf_v, shared.at[pl.ds(s * _NSH + j * 128, 128)])
    pltpu.sync_copy(buf_v.at[pl.ds(0, 120)],
                    shared.at[pl.ds(s * _NSH + 512, 120)])


def _deg_body(dst_hbm, out_hbm, accS, buf_v, idx_v):
    c = lax.axis_index("c")
    s = lax.axis_index("s")
    wid = s * NC + c

    def _fill(val):
        def _row(r, _):
            buf_v[r, :] = jnp.full((_DW,), val, jnp.float32)
            return 0
        lax.fori_loop(0, _CH1, _row, 0)

    _fill(0.0)
    _zero_shard(buf_v, accS, s, _DW)
    _fill(1.0)
    plsc.subcore_barrier()

    def _chunk(i, _):
        off = wid * _EPW1 + i * _CH1
        pltpu.sync_copy(dst_hbm.at[pl.ds(off, _CH1)], idx_v)
        pltpu.sync_copy(buf_v, accS.at[idx_v], add=True)
        return 0

    lax.fori_loop(0, _CPW1, _chunk, 0)
    plsc.subcore_barrier()
    pltpu.sync_copy(accS.at[pl.ds(s * _NSH, _NSH)],
                    out_hbm.at[c, pl.ds(s * _NSH, _NSH)])


_deg_call = functools.partial(
    pl.kernel,
    out_type=jax.ShapeDtypeStruct((NC, N_PAD, _DW), jnp.float32),
    mesh=plsc.VectorSubcoreMesh(core_axis_name="c", subcore_axis_name="s",
                                num_cores=NC, num_subcores=NS),
    compiler_params=pltpu.CompilerParams(use_tc_tiling_on_sc=False),
    scratch_types=[
        pltpu.VMEM_SHARED((N_PAD + 1, _DW), jnp.float32),
        pltpu.VMEM((_CH1, _DW), jnp.float32),
        pltpu.VMEM((_CH1,), jnp.int32),
    ],
)


_B2 = 1000


def _proj_body(deg_ref, x_ref, w_ref, dinv_ref, p_ref):
    degs = deg_ref[0, :, 0:1] + deg_ref[1, :, 0:1] + 1.0
    dinv = lax.rsqrt(degs)
    dinv_ref[...] = dinv
    p = jnp.dot(x_ref[...], w_ref[...], preferred_element_type=jnp.float32)
    p_ref[...] = p * dinv


def _proj(degT, xflat, wbig):
    return pl.pallas_call(
        _proj_body,
        grid=(N // _B2,),
        in_specs=[
            pl.BlockSpec((NC, _B2, _DW), lambda i: (0, i, 0)),
            pl.BlockSpec((_B2, F_IN * T), lambda i: (i, 0)),
            pl.BlockSpec((F_IN * T, 3 * OUT * T), lambda i: (0, 0)),
        ],
        out_specs=[
            pl.BlockSpec((_B2, 1), lambda i: (i, 0)),
            pl.BlockSpec((_B2, 3 * OUT * T), lambda i: (i, 0)),
        ],
        out_shape=[
            jax.ShapeDtypeStruct((N, 1), jnp.float32),
            jax.ShapeDtypeStruct((N, 3 * OUT * T), jnp.float32),
        ],
    )(degT, xflat, wbig)


_CH3 = 128
_CPT = 157
_EPT = _CH3 * _CPT
_EP = _EPT * NS
_TPC = T // NC


def _agg_body(ps_hbm, src_hbm, dst_hbm, out_hbm,
              sS, idx_s, idx_g, idx_d, rows, zbuf):
    c = lax.axis_index("c")
    s = lax.axis_index("s")

    def _zrow(r, _):
        for k in range(3 * OUT // 16):
            zbuf[r, pl.ds(k * 16, 16)] = jnp.zeros((16,), jnp.float32)
        return 0

    lax.fori_loop(0, _CH3, _zrow, 0)
    _zero_shard(zbuf, sS, s, 3 * OUT)
    plsc.subcore_barrier()

    for u in range(_TPC):
        t = c * _TPC + u

        def _chunk(i, _):
            off = s * _EPT + i * _CH3
            pltpu.sync_copy(src_hbm.at[pl.ds(off, _CH3)], idx_s)
            for j in range(_CH3 // 16):
                v = idx_s[pl.ds(j * 16, 16)]
                idx_g[pl.ds(j * 16, 16)] = v * T + t
            pltpu.sync_copy(ps_hbm.at[idx_g], rows)
            pltpu.sync_copy(dst_hbm.at[pl.ds(off, _CH3)], idx_d)
            pltpu.sync_copy(rows, sS.at[idx_d], add=True)
            return 0

        lax.fori_loop(0, _CPT, _chunk, 0)
        plsc.subcore_barrier()
        pltpu.sync_copy(sS.at[pl.ds(s * _NSH, _NSH)],
                        out_hbm.at[t, pl.ds(s * _NSH, _NSH)])
        if u < _TPC - 1:
            _zero_shard(zbuf, sS, s, 3 * OUT)
            plsc.subcore_barrier()


_agg_call = functools.partial(
    pl.kernel,
    out_type=jax.ShapeDtypeStruct((T, N_PAD, 3 * OUT), jnp.float32),
    mesh=plsc.VectorSubcoreMesh(core_axis_name="c", subcore_axis_name="s",
                                num_cores=NC, num_subcores=NS),
    compiler_params=pltpu.CompilerParams(use_tc_tiling_on_sc=False),
    scratch_types=[
        pltpu.VMEM_SHARED((N_PAD + 1, 3 * OUT), jnp.float32),
        pltpu.VMEM((_CH3,), jnp.int32),
        pltpu.VMEM((_CH3,), jnp.int32),
        pltpu.VMEM((_CH3,), jnp.int32),
        pltpu.VMEM((_CH3, 3 * OUT), jnp.float32),
        pltpu.VMEM((_CH3, 3 * OUT), jnp.float32),
    ],
)


_B4 = 1000


def _rec_body(s_ref, p_ref, dinv_ref, lzw_ref, lzb_ref, lrw_ref, lrb_ref,
              lhw_ref, lhb_ref, bz_ref, br_ref, bh_ref, att_ref, wl_ref,
              bl_ref, out_ref):
    a = att_ref[...]
    e = jnp.exp(a - jnp.max(a, axis=1, keepdims=True))
    w = jnp.mean(e / jnp.sum(e, axis=1, keepdims=True), axis=0,
                 keepdims=True)
    dcol = dinv_ref[...]
    pb = p_ref[...]
    lzw = lzw_ref[...]
    lrw = lrw_ref[...]
    lhw = lhw_ref[...]
    H = jnp.zeros((_B4, OUT), jnp.float32)
    Hacc = jnp.zeros((_B4, OUT), jnp.float32)
    for t in range(T):
        g = (s_ref[t] + pb[:, t * 48:(t + 1) * 48]) * dcol
        gz = g[:, 0:16] + bz_ref[...]
        gr = g[:, 16:32] + br_ref[...]
        gh = g[:, 32:48] + bh_ref[...]
        z = jax.nn.sigmoid(
            jnp.dot(gz, lzw[0:16], preferred_element_type=jnp.float32)
            + jnp.dot(H, lzw[16:32], preferred_element_type=jnp.float32)
            + lzb_ref[...])
        r = jax.nn.sigmoid(
            jnp.dot(gr, lrw[0:16], preferred_element_type=jnp.float32)
            + jnp.dot(H, lrw[16:32], preferred_element_type=jnp.float32)
            + lrb_ref[...])
        hh = jnp.tanh(
            jnp.dot(gh, lhw[0:16], preferred_element_type=jnp.float32)
            + jnp.dot(r * H, lhw[16:32], preferred_element_type=jnp.float32)
            + lhb_ref[...])
        H = z * H + (1.0 - z) * hh
        Hacc = Hacc + w[0, t] * H
    out_ref[...] = (jnp.dot(jnp.maximum(Hacc, 0.0), wl_ref[...],
                            preferred_element_type=jnp.float32)
                    + bl_ref[...])


def _recur(S, P, dinv, LzW, Lzb, LrW, Lrb, LhW, Lhb, bz, br, bh, att, WL, bL):
    full = lambda shape: pl.BlockSpec(shape, lambda i: tuple(0 for _ in shape))
    return pl.pallas_call(
        _rec_body,
        grid=(N // _B4,),
        in_specs=[
            pl.BlockSpec((T, _B4, 3 * OUT), lambda i: (0, i, 0)),
            pl.BlockSpec((_B4, T * 3 * OUT), lambda i: (i, 0)),
            pl.BlockSpec((_B4, 1), lambda i: (i, 0)),
            full((2 * OUT, OUT)), full((1, OUT)),
            full((2 * OUT, OUT)), full((1, OUT)),
            full((2 * OUT, OUT)), full((1, OUT)),
            full((1, OUT)), full((1, OUT)), full((1, OUT)),
            full((4, T)), full((OUT, T)), full((1, T)),
        ],
        out_specs=pl.BlockSpec((_B4, T), lambda i: (i, 0)),
        out_shape=jax.ShapeDtypeStruct((N, T), jnp.float32),
    )(S, P, dinv, LzW, Lzb.reshape(1, OUT), LrW, Lrb.reshape(1, OUT),
      LhW, Lhb.reshape(1, OUT), bz.reshape(1, OUT), br.reshape(1, OUT),
      bh.reshape(1, OUT), att, WL, bL.reshape(1, T))


def kernel(x, edge_index, Wz, bz, Wr, br, Wh, bh, LzW, Lzb, LrW, Lrb,
           LhW, Lhb, att, WL, bL):
    src = edge_index[0]
    dst = edge_index[1]
    srcp = jnp.concatenate([src, jnp.zeros((_EP - E,), jnp.int32)])
    dstp = jnp.concatenate([dst, jnp.full((_EP1 - E,), _TRASH, jnp.int32)])

    deg2 = _deg_call(_deg_body)(dstp)

    wcat = jnp.concatenate([Wz, Wr, Wh], axis=1)
    eye = jnp.eye(T, dtype=jnp.float32)
    wbig = (wcat[:, None, None, :] * eye[None, :, :, None]).reshape(
        F_IN * T, T * 3 * OUT)
    dinv, P = _proj(deg2, x.reshape(N, F_IN * T), wbig)

    S = _agg_call(_agg_body)(P.reshape(N * T, 3 * OUT), srcp, dstp)

    return _recur(S, P, dinv, LzW, Lzb, LrW, Lrb, LhW, Lhb,
                  bz, br, bh, att, WL, bL)


# speedup vs baseline: 41.2003x; 1.6424x over previous
import functools

import jax
import jax.numpy as jnp
from jax import lax
from jax.experimental import pallas as pl
from jax.experimental.pallas import tpu as pltpu
import jax.experimental.pallas.tpu_sc as plsc

N = 10000
F_IN = 128
OUT = 16
T = 12
E = 320000

NC = 2
NS = 16
NW = NC * NS

_NSH = 632
N_PAD = _NSH * NS
_TRASH = N_PAD

_CH1 = 128
_CPW1 = 79
_EPW1 = _CH1 * _CPW1
_EP1 = _EPW1 * NW
_DW = 16


def _zero_shard(buf_v, shared, s, width):
    del width
    for j in range(4):
        pltpu.sync_copy(buf_v, shared.at[pl.ds(s * _NSH + j * 128, 128)])
    pltpu.sync_copy(buf_v.at[pl.ds(0, 120)],
                    shared.at[pl.ds(s * _NSH + 512, 120)])


def _deg_body(dst_hbm, out_hbm, accS, buf_v, idx_v):
    c = lax.axis_index("c")
    s = lax.axis_index("s")
    wid = s * NC + c

    def _fill(val):
        def _row(r, _):
            buf_v[r, :] = jnp.full((_DW,), val, jnp.float32)
            return 0
        lax.fori_loop(0, _CH1, _row, 0)

    _fill(0.0)
    _zero_shard(buf_v, accS, s, _DW)
    _fill(1.0)
    pltpu.sync_copy(dst_hbm.at[pl.ds(wid * _CPW1, _CPW1)], idx_v)
    plsc.subcore_barrier()

    def _chunk(i, _):
        pltpu.sync_copy(buf_v, accS.at[idx_v.at[i]], add=True)
        return 0

    lax.fori_loop(0, _CPW1, _chunk, 0)
    plsc.subcore_barrier()
    pltpu.sync_copy(accS.at[pl.ds(s * _NSH, _NSH)],
                    out_hbm.at[c, pl.ds(s * _NSH, _NSH)])


_deg_call = functools.partial(
    pl.kernel,
    out_type=jax.ShapeDtypeStruct((NC, N_PAD, _DW), jnp.float32),
    mesh=plsc.VectorSubcoreMesh(core_axis_name="c", subcore_axis_name="s",
                                num_cores=NC, num_subcores=NS),
    compiler_params=pltpu.CompilerParams(use_tc_tiling_on_sc=False),
    scratch_types=[
        pltpu.VMEM_SHARED((N_PAD + 1, _DW), jnp.float32),
        pltpu.VMEM((_CH1, _DW), jnp.float32),
        pltpu.VMEM((_CPW1, _CH1), jnp.int32),
    ],
)


_B2 = 1000


def _proj_body(deg_ref, x_ref, w_ref, dinv_ref, p_ref):
    degs = deg_ref[0, :, 0:1] + deg_ref[1, :, 0:1] + 1.0
    dinv = lax.rsqrt(degs)
    dinv_ref[...] = dinv
    p = jnp.dot(x_ref[...], w_ref[...], preferred_element_type=jnp.float32)
    p_ref[...] = p * dinv


def _proj(degT, xflat, wbig):
    return pl.pallas_call(
        _proj_body,
        grid=(N // _B2,),
        in_specs=[
            pl.BlockSpec((NC, _B2, _DW), lambda i: (0, i, 0)),
            pl.BlockSpec((_B2, F_IN * T), lambda i: (i, 0)),
            pl.BlockSpec((F_IN * T, 3 * OUT * T), lambda i: (0, 0)),
        ],
        out_specs=[
            pl.BlockSpec((_B2, 1), lambda i: (i, 0)),
            pl.BlockSpec((_B2, 3 * OUT * T), lambda i: (i, 0)),
        ],
        out_shape=[
            jax.ShapeDtypeStruct((N, 1), jnp.float32),
            jax.ShapeDtypeStruct((N, 3 * OUT * T), jnp.float32),
        ],
    )(degT, xflat, wbig)


_CH3 = 128
_CPT = _CPW1 * NC
_EPT = _CH3 * _CPT
_TPC = T // NC


def _agg_body(ps_hbm, src_hbm, dst_hbm, out_hbm,
              sS, src2d, dst2d, idxg, rows0, rows1, zbuf,
              sem_g0, sem_g1, sem_s0, sem_s1):
    c = lax.axis_index("c")
    s = lax.axis_index("s")

    def _zrow(r, _):
        for k in range(3 * OUT // 16):
            zbuf[r, pl.ds(k * 16, 16)] = jnp.zeros((16,), jnp.float32)
        return 0

    lax.fori_loop(0, _CH3, _zrow, 0)
    _zero_shard(zbuf, sS, s, 3 * OUT)
    pltpu.sync_copy(src_hbm.at[pl.ds(s * _CPT, _CPT)], src2d)
    pltpu.sync_copy(dst_hbm.at[pl.ds(s * _CPT, _CPT)], dst2d)
    plsc.subcore_barrier()

    for u in range(_TPC):
        t = c * _TPC + u

        def _xform(i, _):
            for j in range(_CH3 // 16):
                v = src2d[i, pl.ds(j * 16, 16)]
                idxg[i, pl.ds(j * 16, 16)] = v * T + t
            return 0

        lax.fori_loop(0, _CPT, _xform, 0)

        def _group(g, _):
            c0 = 2 * g
            c1 = 2 * g + 1
            d0 = pltpu.async_copy(ps_hbm.at[idxg.at[c0]], rows0, sem_g0)
            d1 = pltpu.async_copy(ps_hbm.at[idxg.at[c1]], rows1, sem_g1)
            d0.wait()
            s0 = pltpu.async_copy(rows0, sS.at[dst2d.at[c0]], sem_s0,
                                  add=True)
            d1.wait()
            s1 = pltpu.async_copy(rows1, sS.at[dst2d.at[c1]], sem_s1,
                                  add=True)
            s0.wait()
            s1.wait()
            return 0

        lax.fori_loop(0, _CPT // 2, _group, 0)
        plsc.subcore_barrier()
        pltpu.sync_copy(sS.at[pl.ds(s * _NSH, _NSH)],
                        out_hbm.at[t, pl.ds(s * _NSH, _NSH)])
        if u < _TPC - 1:
            _zero_shard(zbuf, sS, s, 3 * OUT)
            plsc.subcore_barrier()


_agg_call = functools.partial(
    pl.kernel,
    out_type=jax.ShapeDtypeStruct((T, N_PAD, 3 * OUT), jnp.float32),
    mesh=plsc.VectorSubcoreMesh(core_axis_name="c", subcore_axis_name="s",
                                num_cores=NC, num_subcores=NS),
    compiler_params=pltpu.CompilerParams(use_tc_tiling_on_sc=False),
    scratch_types=[
        pltpu.VMEM_SHARED((N_PAD + 1, 3 * OUT), jnp.float32),
        pltpu.VMEM((_CPT, _CH3), jnp.int32),
        pltpu.VMEM((_CPT, _CH3), jnp.int32),
        pltpu.VMEM((_CPT, _CH3), jnp.int32),
        pltpu.VMEM((_CH3, 3 * OUT), jnp.float32),
        pltpu.VMEM((_CH3, 3 * OUT), jnp.float32),
        pltpu.VMEM((_CH3, 3 * OUT), jnp.float32),
        pltpu.SemaphoreType.DMA,
        pltpu.SemaphoreType.DMA,
        pltpu.SemaphoreType.DMA,
        pltpu.SemaphoreType.DMA,
    ],
)


_B4 = 1000


def _rec_body(s_ref, p_ref, dinv_ref, lzw_ref, lzb_ref, lrw_ref, lrb_ref,
              lhw_ref, lhb_ref, bz_ref, br_ref, bh_ref, att_ref, wl_ref,
              bl_ref, out_ref):
    a = att_ref[...]
    e = jnp.exp(a - jnp.max(a, axis=1, keepdims=True))
    w = jnp.mean(e / jnp.sum(e, axis=1, keepdims=True), axis=0,
                 keepdims=True)
    dcol = dinv_ref[...]
    pb = p_ref[...]
    lzw = lzw_ref[...]
    lrw = lrw_ref[...]
    lhw = lhw_ref[...]
    H = jnp.zeros((_B4, OUT), jnp.float32)
    Hacc = jnp.zeros((_B4, OUT), jnp.float32)
    for t in range(T):
        g = (s_ref[t] + pb[:, t * 48:(t + 1) * 48]) * dcol
        gz = g[:, 0:16] + bz_ref[...]
        gr = g[:, 16:32] + br_ref[...]
        gh = g[:, 32:48] + bh_ref[...]
        z = jax.nn.sigmoid(
            jnp.dot(gz, lzw[0:16], preferred_element_type=jnp.float32)
            + jnp.dot(H, lzw[16:32], preferred_element_type=jnp.float32)
            + lzb_ref[...])
        r = jax.nn.sigmoid(
            jnp.dot(gr, lrw[0:16], preferred_element_type=jnp.float32)
            + jnp.dot(H, lrw[16:32], preferred_element_type=jnp.float32)
            + lrb_ref[...])
        hh = jnp.tanh(
            jnp.dot(gh, lhw[0:16], preferred_element_type=jnp.float32)
            + jnp.dot(r * H, lhw[16:32], preferred_element_type=jnp.float32)
            + lhb_ref[...])
        H = z * H + (1.0 - z) * hh
        Hacc = Hacc + w[0, t] * H
    out_ref[...] = (jnp.dot(jnp.maximum(Hacc, 0.0), wl_ref[...],
                            preferred_element_type=jnp.float32)
                    + bl_ref[...])


def _recur(S, P, dinv, LzW, Lzb, LrW, Lrb, LhW, Lhb, bz, br, bh, att, WL, bL):
    full = lambda shape: pl.BlockSpec(shape, lambda i: tuple(0 for _ in shape))
    return pl.pallas_call(
        _rec_body,
        grid=(N // _B4,),
        in_specs=[
            pl.BlockSpec((T, _B4, 3 * OUT), lambda i: (0, i, 0)),
            pl.BlockSpec((_B4, T * 3 * OUT), lambda i: (i, 0)),
            pl.BlockSpec((_B4, 1), lambda i: (i, 0)),
            full((2 * OUT, OUT)), full((1, OUT)),
            full((2 * OUT, OUT)), full((1, OUT)),
            full((2 * OUT, OUT)), full((1, OUT)),
            full((1, OUT)), full((1, OUT)), full((1, OUT)),
            full((4, T)), full((OUT, T)), full((1, T)),
        ],
        out_specs=pl.BlockSpec((_B4, T), lambda i: (i, 0)),
        out_shape=jax.ShapeDtypeStruct((N, T), jnp.float32),
    )(S, P, dinv, LzW, Lzb.reshape(1, OUT), LrW, Lrb.reshape(1, OUT),
      LhW, Lhb.reshape(1, OUT), bz.reshape(1, OUT), br.reshape(1, OUT),
      bh.reshape(1, OUT), att, WL, bL.reshape(1, T))


def kernel(x, edge_index, Wz, bz, Wr, br, Wh, bh, LzW, Lzb, LrW, Lrb,
           LhW, Lhb, att, WL, bL):
    src = edge_index[0]
    dst = edge_index[1]
    srcp = jnp.concatenate(
        [src, jnp.zeros((_EP1 - E,), jnp.int32)]).reshape(-1, _CH1)
    dstp = jnp.concatenate(
        [dst, jnp.full((_EP1 - E,), _TRASH, jnp.int32)]).reshape(-1, _CH1)

    deg2 = _deg_call(_deg_body)(dstp)

    wcat = jnp.concatenate([Wz, Wr, Wh], axis=1)
    eye = jnp.eye(T, dtype=jnp.float32)
    wbig = (wcat[:, None, None, :] * eye[None, :, :, None]).reshape(
        F_IN * T, T * 3 * OUT)
    dinv, P = _proj(deg2, x.reshape(N, F_IN * T), wbig)

    S = _agg_call(_agg_body)(P.reshape(N * T, 3 * OUT), srcp, dstp)

    return _recur(S, P, dinv, LzW, Lzb, LrW, Lrb, LhW, Lhb,
                  bz, br, bh, att, WL, bL)
